# baseline (speedup 1.0000x reference)
import functools

import jax
import jax.numpy as jnp
from jax import lax
from jax.experimental import pallas as pl
from jax.experimental.pallas import tpu as pltpu
from jax.experimental.pallas import tpu_sc as plsc

_RES = 1000000
_LAT = 64
_N = 1048576
_NC = 2
_NS = 16
_NW = _NC * _NS
_BW = _N // _NW
_C = 512
_G = _BW // _C
_SUB = _C // 128


def _body(u_hbm, table_hbm, out_hbm,
          u0, u1, ia0, ia1, ib0, ib1, rows_a, rows_b, out_t,
          sg0, sg1, sg2, sg3, sem_o):
    u_v = (u0, u1)
    idx_a = (ia0, ia1)
    idx_b = (ib0, ib1)
    sem_g = (sg0, sg1, sg2, sg3)
    wid = lax.axis_index("s") * _NC + lax.axis_index("c")
    base0 = wid * _BW
    scale = jnp.float32(_RES - 1)
    lane = lax.iota(jnp.int32, 16)

    def ga_copy(s, j):
        return pltpu.make_async_copy(
            table_hbm.at[idx_a[s].at[j]],
            rows_a.at[pl.ds(j * 128, 128)], sem_g[j])

    def gb_copy(s, j):
        return pltpu.make_async_copy(
            table_hbm.at[idx_b[s].at[j]],
            rows_b.at[pl.ds(j * 128, 128)], sem_g[j])

    def out_copy(g):
        return pltpu.make_async_copy(
            out_t, out_hbm.at[:, pl.ds(base0 + g * _C, _C)], sem_o)

    def prep_meta(g, s):
        base = base0 + g * _C
        pltpu.sync_copy(u_hbm.at[pl.ds(base, _C)], u_v[s])

        def idx_body(j, c):
            for k in range(8):
                off = j * 128 + k * 16
                u16 = u_v[s][pl.ds(off, 16)]
                f = u16 * scale
                ix = f.astype(jnp.int32)
                fl = ix.astype(jnp.float32)
                idx_a[s][j, pl.ds(k * 16, 16)] = ix
                idx_b[s][j, pl.ds(k * 16, 16)] = ix + 1
                u_v[s][pl.ds(off, 16)] = f - fl
            return c

        lax.fori_loop(0, _SUB, idx_body, 0, unroll=False)

    def lerp(s):
        def lerp_body(blk, c):
            i0 = blk * 16
            al16 = u_v[s][pl.ds(i0, 16)]
            for l in range(16):
                i = i0 + l
                al = jnp.full((16,), al16[l], jnp.float32)
                col = jnp.full((16,), i, jnp.int32)
                for r in range(4):
                    a = rows_a[i, pl.ds(r * 16, 16)]
                    bb = rows_b[i, pl.ds(r * 16, 16)]
                    ov = a + al * (bb - a)
                    plsc.store_scatter(out_t, [r * 16 + lane, col], ov)
            return c

        lax.fori_loop(0, _C // 16, lerp_body, 0, unroll=False)

    def chunk(g, s, first):
        @pl.when(g + 1 < _G)
        def _():
            prep_meta(g + 1, 1 - s)

        for j in range(_SUB):
            ga_copy(s, j).wait()
            gb_copy(s, j).wait()
        if not first:
            out_copy(g).wait()
        lerp(s)
        out_copy(g).start()

        @pl.when(g + 1 < _G)
        def _():
            for j in range(_SUB):
                ga_copy(1 - s, j).start()
                gb_copy(1 - s, j).start()

    prep_meta(0, 0)
    for j in range(_SUB):
        ga_copy(0, j).start()
        gb_copy(0, j).start()

    def pair(t, carry):
        chunk(2 * t, 0, False)
        chunk(2 * t + 1, 1, False)
        return carry

    chunk(0, 0, True)
    chunk(1, 1, False)
    lax.fori_loop(1, _G // 2, pair, 0, unroll=False)
    out_copy(_G - 1).wait()


def kernel(uList, table):
    mesh = plsc.VectorSubcoreMesh(core_axis_name="c", subcore_axis_name="s")
    k = functools.partial(
        pl.kernel,
        mesh=mesh,
        out_type=jax.ShapeDtypeStruct((_LAT, _N), jnp.float32),
        compiler_params=pltpu.CompilerParams(
            use_tc_tiling_on_sc=False, needs_layout_passes=False),
        scratch_types=[
            pltpu.VMEM((_C,), jnp.float32),
            pltpu.VMEM((_C,), jnp.float32),
            pltpu.VMEM((_SUB, 128), jnp.int32),
            pltpu.VMEM((_SUB, 128), jnp.int32),
            pltpu.VMEM((_SUB, 128), jnp.int32),
            pltpu.VMEM((_SUB, 128), jnp.int32),
            pltpu.VMEM((_C, _LAT), jnp.float32),
            pltpu.VMEM((_C, _LAT), jnp.float32),
            pltpu.VMEM((_LAT, _C), jnp.float32),
            pltpu.SemaphoreType.DMA,
            pltpu.SemaphoreType.DMA,
            pltpu.SemaphoreType.DMA,
            pltpu.SemaphoreType.DMA,
            pltpu.SemaphoreType.DMA,
        ],
    )(_body)
    out_t = k(uList, table)
    return out_t.T


# speedup vs baseline: 4.9667x; 4.9667x over previous
import functools

import jax
import jax.numpy as jnp
from jax import lax
from jax.experimental import pallas as pl
from jax.experimental.pallas import tpu as pltpu
from jax.experimental.pallas import tpu_sc as plsc

_RES = 1000000
_LAT = 64
_N = 1048576
_NC = 2
_NS = 16
_NW = _NC * _NS
_BW = _N // _NW
_C = 512
_G = _BW // _C
_SUB = _C // 128


def _body(u_hbm, table_hbm, out_hbm,
          u0, u1, ia0, ia1, ib0, ib1, rows_a, rows_b,
          sg0, sg1, sg2, sg3, sem_o):
    u_v = (u0, u1)
    idx_a = (ia0, ia1)
    idx_b = (ib0, ib1)
    sem_g = (sg0, sg1, sg2, sg3)
    wid = lax.axis_index("s") * _NC + lax.axis_index("c")
    base0 = wid * _BW
    scale = jnp.float32(_RES - 1)

    def ga_copy(s, j):
        return pltpu.make_async_copy(
            table_hbm.at[idx_a[s].at[j]],
            rows_a.at[pl.ds(j * 128, 128)], sem_g[j])

    def gb_copy(s, j):
        return pltpu.make_async_copy(
            table_hbm.at[idx_b[s].at[j]],
            rows_b.at[pl.ds(j * 128, 128)], sem_g[j])

    def out_copy(g, j):
        return pltpu.make_async_copy(
            rows_b.at[pl.ds(j * 128, 128)],
            out_hbm.at[pl.ds(base0 + g * _C + j * 128, 128)], sem_o)

    def prep_meta(g, s):
        base = base0 + g * _C
        pltpu.sync_copy(u_hbm.at[pl.ds(base, _C)], u_v[s])

        def idx_body(j, c):
            for k in range(8):
                off = j * 128 + k * 16
                u16 = u_v[s][pl.ds(off, 16)]
                f = u16 * scale
                ix = f.astype(jnp.int32)
                fl = ix.astype(jnp.float32)
                idx_a[s][j, pl.ds(k * 16, 16)] = ix
                idx_b[s][j, pl.ds(k * 16, 16)] = ix + 1
                u_v[s][pl.ds(off, 16)] = f - fl
            return c

        lax.fori_loop(0, _SUB, idx_body, 0, unroll=True)

    def lerp_sub(s, j):
        def lerp_body(blk, c):
            i0 = j * 128 + blk * 16
            al16 = u_v[s][pl.ds(i0, 16)]
            for l in range(16):
                al = jnp.full((16,), al16[l], jnp.float32)
                for r in range(4):
                    a = rows_a[i0 + l, pl.ds(r * 16, 16)]
                    bb = rows_b[i0 + l, pl.ds(r * 16, 16)]
                    rows_b[i0 + l, pl.ds(r * 16, 16)] = a + al * (bb - a)
            return c

        lax.fori_loop(0, 8, lerp_body, 0, unroll=False)

    def chunk(g, s, last):
        if not last:
            prep_meta(g + 1, 1 - s)
        for j in range(_SUB):
            ga_copy(s, j).wait()
            gb_copy(s, j).wait()
            lerp_sub(s, j)
            out_copy(g, j).start()
        if not last:
            for j in range(_SUB):
                ga_copy(1 - s, j).start()
        for j in range(_SUB):
            out_copy(g, j).wait()
        if not last:
            for j in range(_SUB):
                gb_copy(1 - s, j).start()

    prep_meta(0, 0)
    for j in range(_SUB):
        ga_copy(0, j).start()
        gb_copy(0, j).start()

    def pair(t, carry):
        chunk(2 * t, 0, False)
        chunk(2 * t + 1, 1, False)
        return carry

    lax.fori_loop(0, _G // 2 - 1, pair, 0, unroll=False)

    chunk(_G - 2, 0, False)
    chunk(_G - 1, 1, True)


def kernel(uList, table):
    mesh = plsc.VectorSubcoreMesh(core_axis_name="c", subcore_axis_name="s")
    k = functools.partial(
        pl.kernel,
        mesh=mesh,
        out_type=jax.ShapeDtypeStruct((_N, _LAT), jnp.float32),
        compiler_params=pltpu.CompilerParams(use_tc_tiling_on_sc=False),
        scratch_types=[
            pltpu.VMEM((_C,), jnp.float32),
            pltpu.VMEM((_C,), jnp.float32),
            pltpu.VMEM((_SUB, 128), jnp.int32),
            pltpu.VMEM((_SUB, 128), jnp.int32),
            pltpu.VMEM((_SUB, 128), jnp.int32),
            pltpu.VMEM((_SUB, 128), jnp.int32),
            pltpu.VMEM((_C, _LAT), jnp.float32),
            pltpu.VMEM((_C, _LAT), jnp.float32),
            pltpu.SemaphoreType.DMA,
            pltpu.SemaphoreType.DMA,
            pltpu.SemaphoreType.DMA,
            pltpu.SemaphoreType.DMA,
            pltpu.SemaphoreType.DMA,
        ],
    )(_body)
    return k(uList, table)
